# Initial kernel scaffold; baseline (speedup 1.0000x reference)
import jax
import jax.numpy as jnp
from jax.experimental import pallas as pl


def kernel(x, edge_index, batch, Wt, bt, W0, as0, ad0, bb0, W1, as1, ad1, bb1, W2, as2, ad2, bb2, g0, be0, g1, be1, g2, be2, Wm1, bm1, Wm2, bm2):
    raise NotImplementedError("write your pallas kernel here")


# speedup vs baseline: 1.0309x; 1.0309x over previous
import functools
import jax
import jax.numpy as jnp
from jax.experimental import pallas as pl
from jax.experimental.pallas import tpu as pltpu

N_NODES = 10000
N_EDGES = 160000
N_GRAPHS = 128
EMB = 780


def _bf16x3_dot(x, w):
    xh = x.astype(jnp.bfloat16)
    xl = (x - xh.astype(jnp.float32)).astype(jnp.bfloat16)
    wh = w.astype(jnp.bfloat16)
    wl = (w - wh.astype(jnp.float32)).astype(jnp.bfloat16)
    f32 = jnp.float32
    cross = (jnp.dot(xh, wl, preferred_element_type=f32)
             + jnp.dot(xl, wh, preferred_element_type=f32))
    return jnp.dot(xh, wh, preferred_element_type=f32) + cross


def _mm_attn_body(heads, x_ref, w_ref, hb_ref, asd_ref, h_ref, aa_ref):
    h = jnp.dot(x_ref[...].astype(jnp.bfloat16),
                w_ref[...].astype(jnp.bfloat16),
                preferred_element_type=jnp.float32) + hb_ref[...]
    h_ref[...] = h
    cols = []
    for j in range(2 * heads):
        seg = j % heads
        vec = asd_ref[j:j + 1, :]
        cols.append(jnp.sum(h[:, seg * EMB:(seg + 1) * EMB] * vec,
                            axis=1, keepdims=True))
    blk = h.shape[0]
    cols.append(jnp.zeros((blk, 8 - 2 * heads), jnp.float32))
    aa_ref[...] = jnp.concatenate(cols, axis=1)


def _mm_attn(x, w, hb, asd, heads, block_n=400):
    n, din = x.shape
    dout = w.shape[1]
    grid = (n // block_n,)
    return pl.pallas_call(
        functools.partial(_mm_attn_body, heads),
        grid=grid,
        in_specs=[
            pl.BlockSpec((block_n, din), lambda i: (i, 0)),
            pl.BlockSpec((din, dout), lambda i: (0, 0)),
            pl.BlockSpec((1, dout), lambda i: (0, 0)),
            pl.BlockSpec((2 * heads, EMB), lambda i: (0, 0)),
        ],
        out_specs=[
            pl.BlockSpec((block_n, dout), lambda i: (i, 0)),
            pl.BlockSpec((block_n, 8), lambda i: (i, 0)),
        ],
        out_shape=[
            jax.ShapeDtypeStruct((n, dout), jnp.float32),
            jax.ShapeDtypeStruct((n, 8), jnp.float32),
        ],
    )(x, w, hb, asd)


def _tanh_bn_body(agg_ref, b_ref, g_ref, be_ref, o_ref):
    y = jnp.tanh(agg_ref[...] + b_ref[...])
    m = jnp.mean(y, axis=0, keepdims=True)
    v = jnp.mean((y - m) ** 2, axis=0, keepdims=True)
    o_ref[...] = (y - m) * jax.lax.rsqrt(v + 1e-5) * g_ref[...] + be_ref[...]


def _tanh_bn(agg, b, g, be, block_c=128):
    n, d0 = agg.shape
    d = ((d0 + block_c - 1) // block_c) * block_c
    if d != d0:
        agg = jnp.pad(agg, ((0, 0), (0, d - d0)))
        b = jnp.pad(b, (0, d - d0))
        g = jnp.pad(g, (0, d - d0))
        be = jnp.pad(be, (0, d - d0))
    grid = (d // block_c,)
    b2 = b.reshape(1, d)
    g2 = g.reshape(1, d)
    be2 = be.reshape(1, d)
    out = pl.pallas_call(
        _tanh_bn_body,
        grid=grid,
        in_specs=[
            pl.BlockSpec((n, block_c), lambda i: (0, i)),
            pl.BlockSpec((1, block_c), lambda i: (0, i)),
            pl.BlockSpec((1, block_c), lambda i: (0, i)),
            pl.BlockSpec((1, block_c), lambda i: (0, i)),
        ],
        out_specs=pl.BlockSpec((n, block_c), lambda i: (0, i)),
        out_shape=jax.ShapeDtypeStruct((n, d), jnp.float32),
    )(agg, b2, g2, be2)
    return out[:, :d0] if d != d0 else out


def _pool_mlp_body(x_ref, batch_ref, wm1_ref, bm1_ref, wm2_ref, bm2_ref, o_ref):
    gids = jax.lax.broadcasted_iota(jnp.int32, (N_GRAPHS, N_NODES), 0)
    sel = (batch_ref[...] == gids).astype(jnp.float32)
    s = jnp.dot(sel, x_ref[...], preferred_element_type=jnp.float32, precision=jax.lax.Precision.HIGHEST)
    cnt = jnp.sum(sel, axis=1, keepdims=True)
    pooled = s / jnp.maximum(cnt, 1.0)
    h = jnp.maximum(pooled, 0.0)
    h = jnp.tanh(jnp.dot(h, wm1_ref[...], preferred_element_type=jnp.float32, precision=jax.lax.Precision.HIGHEST)
                 + bm1_ref[...])
    o_ref[...] = jnp.dot(h, wm2_ref[...], preferred_element_type=jnp.float32, precision=jax.lax.Precision.HIGHEST) + bm2_ref[...]


def _pool_mlp(x, batch_row, wm1, bm1, wm2, bm2):
    return pl.pallas_call(
        _pool_mlp_body,
        out_shape=jax.ShapeDtypeStruct((N_GRAPHS, 64), jnp.float32),
    )(x, batch_row, wm1, bm1.reshape(1, -1), wm2, bm2.reshape(1, -1))


def _edge_phase(h, asrc, adst, src, dst, heads):
    e = asrc[src] + adst[dst]
    e = jnp.where(e >= 0, e, 0.2 * e)
    emax = jax.ops.segment_max(e, dst, num_segments=N_NODES)
    emax = jnp.where(jnp.isfinite(emax), emax, 0.0)
    ex = jnp.exp(e - emax[dst])
    den = jax.ops.segment_sum(ex, dst, num_segments=N_NODES)
    alpha = ex / (den[dst] + 1e-16)
    hh = h.reshape(N_NODES, heads, EMB)
    msg = hh[src] * alpha[:, :, None]
    out = jax.ops.segment_sum(msg, dst, num_segments=N_NODES)
    return out.reshape(N_NODES, heads * EMB)


def _attn_mats(a_s, a_d, dout, heads):
    a = jnp.zeros((dout, 8), jnp.float32)
    for k in range(heads):
        a = a.at[k * EMB:(k + 1) * EMB, k].set(a_s[k])
        a = a.at[k * EMB:(k + 1) * EMB, heads + k].set(a_d[k])
    return a


def _gat_layer(x, src, dst, w, hbias, a_s, a_d, b, g, be, heads):
    dout = heads * EMB
    asd = jnp.concatenate([a_s, a_d], axis=0)
    h, aa = _mm_attn(x, w, hbias.reshape(1, dout), asd, heads)
    asrc = aa[:, :heads]
    adst = aa[:, heads:2 * heads]
    agg = _edge_phase(h, asrc, adst, src, dst, heads)
    return _tanh_bn(agg, b, g, be)


def kernel(x, edge_index, batch, Wt, bt, W0, as0, ad0, bb0, W1, as1, ad1, bb1,
           W2, as2, ad2, bb2, g0, be0, g1, be1, g2, be2, Wm1, bm1, Wm2, bm2):
    src = edge_index[0]
    dst = edge_index[1]
    W0p = jnp.concatenate([W0[:768], Wt @ W0[768:780]], axis=0)
    hb0 = bt @ W0[768:780]
    z = jnp.zeros((3 * EMB,), jnp.float32)

    x1 = _gat_layer(x, src, dst, W0p, hb0, as0, ad0, bb0, g0, be0, heads=3)
    x2 = _gat_layer(x1, src, dst, W1, z, as1, ad1, bb1, g1, be1, heads=3)
    x3 = _gat_layer(x2, src, dst, W2, z[:EMB], as2, ad2, bb2, g2, be2, heads=1)

    batch_row = batch.astype(jnp.int32).reshape(1, N_NODES)
    return _pool_mlp(x3, batch_row, Wm1, bm1, Wm2, bm2)
